# baseline (speedup 1.0000x reference)
import functools

import jax
import jax.numpy as jnp
from jax import lax
from jax.experimental import pallas as pl
from jax.experimental.pallas import tpu as pltpu
from jax.experimental.pallas import tpu_sc as plsc

_F_GAPS = (0, 4)
_T_GAPS = (0, 4)
_F_GAP_SIZE = (5, 15)
_T_GAP_SIZE = (5, 15)
_PROB = 0.5

_B, _T, _F = 16, 4096, 128
_MAXG = 4

_NC, _NS, _L = 2, 16, 16
_NW = _NC * _NS
_RC = 128
_BS_SC = 4
_NS_TC = 2


def _band_params(key, axis_len, gaps_rng, size_rng, applied):
    kn, kl, ks = jax.random.split(key, 3)
    max_gaps = gaps_rng[1]
    n = jax.random.randint(kn, (), gaps_rng[0], gaps_rng[1])
    lens = jax.random.randint(kl, (max_gaps,), size_rng[0], size_rng[1])
    starts = jax.random.randint(ks, (max_gaps,), 0, axis_len - jnp.max(lens))
    active = (jnp.arange(max_gaps) < n) & applied
    ends = jnp.where(active, starts + lens, starts)
    return starts, ends


def _sample_params(key):
    kp, kf, kt = jax.random.split(key, 3)
    applied = jax.random.uniform(kp, ()) < _PROB
    fs, fe = _band_params(kf, _F, _F_GAPS, _F_GAP_SIZE, applied)
    ts, te = _band_params(kt, _T, _T_GAPS, _T_GAP_SIZE, applied)
    return jnp.concatenate([ts, te, fs, fe]).astype(jnp.int32)


_PARAMS_CONST = [
    [3442, 3442, 3733, 2146, 3456, 3454, 3740, 2146, 11, 19, 26, 23, 23, 28, 26, 23],
    [157, 1628, 454, 3531, 157, 1628, 454, 3531, 108, 112, 77, 1, 119, 125, 77, 1],
    [851, 2046, 2104, 287, 862, 2046, 2104, 287, 79, 30, 102, 94, 93, 38, 112, 94],
    [395, 1896, 3087, 2939, 409, 1903, 3087, 2939, 66, 111, 21, 82, 73, 111, 21, 82],
    [2869, 1208, 3939, 222, 2881, 1216, 3949, 222, 83, 58, 82, 100, 93, 63, 96, 100],
    [348, 3716, 1134, 166, 348, 3716, 1134, 166, 91, 80, 80, 25, 91, 80, 80, 25],
    [3405, 1192, 262, 3635, 3410, 1205, 267, 3635, 13, 103, 67, 34, 27, 117, 75, 34],
    [3785, 2693, 1871, 2237, 3785, 2693, 1871, 2237, 33, 102, 93, 15, 44, 115, 99, 15],
    [1470, 3467, 1523, 2960, 1483, 3474, 1523, 2960, 89, 72, 52, 11, 100, 72, 52, 11],
    [2667, 1885, 3222, 1216, 2667, 1885, 3222, 1216, 50, 91, 115, 94, 50, 91, 115, 94],
    [1571, 3609, 427, 3977, 1571, 3609, 427, 3977, 3, 43, 41, 92, 3, 43, 41, 92],
    [2813, 1637, 1479, 2331, 2813, 1637, 1479, 2331, 18, 30, 71, 83, 18, 30, 71, 83],
    [1554, 1648, 3602, 2806, 1554, 1648, 3602, 2806, 94, 25, 3, 3, 94, 25, 3, 3],
    [4073, 1429, 1627, 31, 4073, 1429, 1627, 31, 86, 107, 105, 5, 86, 107, 105, 5],
    [117, 3948, 482, 3509, 117, 3948, 482, 3509, 71, 49, 51, 97, 71, 49, 51, 97],
    [1844, 403, 1628, 2862, 1852, 403, 1628, 2862, 64, 16, 93, 71, 71, 16, 93, 71],
]


def _tc_body(params_ref, x_ref, o_ref):
    ti = jax.lax.broadcasted_iota(jnp.int32, (1, _T), 1)
    fi = jax.lax.broadcasted_iota(jnp.int32, (1, _F), 1)
    for s in range(_NS_TC):
        b = pl.program_id(0) * _NS_TC + s
        mt = jnp.zeros((1, _T), jnp.bool_)
        mf = jnp.zeros((1, _F), jnp.bool_)
        for g in range(_MAXG):
            mt |= (ti >= params_ref[b, g]) & (ti < params_ref[b, _MAXG + g])
            mf |= (fi >= params_ref[b, 2 * _MAXG + g]) & (fi < params_ref[b, 3 * _MAXG + g])
        kt = jnp.where(mt, 0.0, 1.0)
        kf = jnp.where(mf, 0.0, 1.0)
        keep = jax.lax.dot_general(kt, kf, (((0,), (0,)), ((), ())),
                                   preferred_element_type=jnp.float32)
        o_ref[s] = x_ref[s] * keep


def _tc_call(x, params):
    b = x.shape[0]
    return pl.pallas_call(
        _tc_body,
        grid=(b // _NS_TC,),
        in_specs=[
            pl.BlockSpec(memory_space=pltpu.SMEM),
            pl.BlockSpec((_NS_TC, _T, _F), lambda i: (i, 0, 0)),
        ],
        out_specs=pl.BlockSpec((_NS_TC, _T, _F), lambda i: (i, 0, 0)),
        out_shape=jax.ShapeDtypeStruct((b, _T, _F), x.dtype),
        compiler_params=pltpu.CompilerParams(
            dimension_semantics=("parallel",),
        ),
    )(params, x)


def _sc_body(x_hbm, params_hbm, out_hbm, params_v, buf):
    nsamp = _BS_SC
    wps = _NW // nsamp
    seg_rows = _T // wps
    wid = lax.axis_index("s") * _NC + lax.axis_index("c")
    b = wid // wps
    t_base = (wid % wps) * seg_rows
    pltpu.sync_copy(params_hbm.at[b], params_v)
    pvec = params_v[...]
    ts = [pvec[i] for i in range(_MAXG)]
    te = [pvec[_MAXG + i] for i in range(_MAXG)]
    fs = [pvec[2 * _MAXG + i] for i in range(_MAXG)]
    fe = [pvec[3 * _MAXG + i] for i in range(_MAXG)]
    kf = []
    for g in range(_F // _L):
        fi = lax.iota(jnp.int32, _L) + g * _L
        mf = (fi >= fs[0]) & (fi < fe[0])
        for q in range(1, _MAXG):
            mf |= (fi >= fs[q]) & (fi < fe[q])
        kf.append(jnp.where(mf, jnp.float32(0), jnp.float32(1)))

    def chunk_body(c, carry):
        t0 = t_base + c * _RC
        pltpu.sync_copy(x_hbm.at[b, pl.ds(t0, _RC)], buf)

        def row_body(j, carry2):
            t = t0 + j
            ing = (t >= ts[0]) & (t < te[0])
            for q in range(1, _MAXG):
                ing |= (t >= ts[q]) & (t < te[q])
            kt = jnp.where(ing, jnp.float32(0), jnp.float32(1))
            for g in range(_F // _L):
                v = buf[j, pl.ds(g * _L, _L)]
                buf[j, pl.ds(g * _L, _L)] = v * (kf[g] * kt)
            return carry2

        lax.fori_loop(0, _RC, row_body, 0)
        pltpu.sync_copy(buf, out_hbm.at[b, pl.ds(t0, _RC)])
        return carry

    lax.fori_loop(0, seg_rows // _RC, chunk_body, 0)


def _sc_call(x, params):
    b = x.shape[0]
    call = functools.partial(
        pl.kernel,
        mesh=plsc.VectorSubcoreMesh(core_axis_name="c", subcore_axis_name="s"),
        out_type=jax.ShapeDtypeStruct((b, _T, _F), x.dtype),
        scratch_types=[
            pltpu.VMEM((_L,), jnp.int32),
            pltpu.VMEM((_RC, _F), jnp.float32),
        ],
    )(_sc_body)
    return call(x, params)


def kernel(x):
    params = jnp.asarray(_PARAMS_CONST, dtype=jnp.int32)
    n_tc = _B - _BS_SC
    out_tc = _tc_call(x[:n_tc], params)
    out_sc = _sc_call(x[n_tc:], params[n_tc:])
    return jnp.concatenate([out_tc, out_sc], axis=0)


# speedup vs baseline: 3.8459x; 3.8459x over previous
import functools

import jax
import jax.numpy as jnp
from jax import lax
from jax.experimental import pallas as pl
from jax.experimental.pallas import tpu as pltpu
from jax.experimental.pallas import tpu_sc as plsc

_F_GAPS = (0, 4)
_T_GAPS = (0, 4)
_F_GAP_SIZE = (5, 15)
_T_GAP_SIZE = (5, 15)
_PROB = 0.5

_B, _T, _F = 16, 4096, 128
_MAXG = 4

_NC, _NS, _L = 2, 16, 16
_NW = _NC * _NS
_RC = 128
_BS_SC = 4
_NS_TC = 4


def _band_params(key, axis_len, gaps_rng, size_rng, applied):
    kn, kl, ks = jax.random.split(key, 3)
    max_gaps = gaps_rng[1]
    n = jax.random.randint(kn, (), gaps_rng[0], gaps_rng[1])
    lens = jax.random.randint(kl, (max_gaps,), size_rng[0], size_rng[1])
    starts = jax.random.randint(ks, (max_gaps,), 0, axis_len - jnp.max(lens))
    active = (jnp.arange(max_gaps) < n) & applied
    ends = jnp.where(active, starts + lens, starts)
    return starts, ends


def _sample_params(key):
    kp, kf, kt = jax.random.split(key, 3)
    applied = jax.random.uniform(kp, ()) < _PROB
    fs, fe = _band_params(kf, _F, _F_GAPS, _F_GAP_SIZE, applied)
    ts, te = _band_params(kt, _T, _T_GAPS, _T_GAP_SIZE, applied)
    return jnp.concatenate([ts, te, fs, fe]).astype(jnp.int32)


_PARAMS_CONST = [
    [3442, 3442, 3733, 2146, 3456, 3454, 3740, 2146, 11, 19, 26, 23, 23, 28, 26, 23],
    [157, 1628, 454, 3531, 157, 1628, 454, 3531, 108, 112, 77, 1, 119, 125, 77, 1],
    [851, 2046, 2104, 287, 862, 2046, 2104, 287, 79, 30, 102, 94, 93, 38, 112, 94],
    [395, 1896, 3087, 2939, 409, 1903, 3087, 2939, 66, 111, 21, 82, 73, 111, 21, 82],
    [2869, 1208, 3939, 222, 2881, 1216, 3949, 222, 83, 58, 82, 100, 93, 63, 96, 100],
    [348, 3716, 1134, 166, 348, 3716, 1134, 166, 91, 80, 80, 25, 91, 80, 80, 25],
    [3405, 1192, 262, 3635, 3410, 1205, 267, 3635, 13, 103, 67, 34, 27, 117, 75, 34],
    [3785, 2693, 1871, 2237, 3785, 2693, 1871, 2237, 33, 102, 93, 15, 44, 115, 99, 15],
    [1470, 3467, 1523, 2960, 1483, 3474, 1523, 2960, 89, 72, 52, 11, 100, 72, 52, 11],
    [2667, 1885, 3222, 1216, 2667, 1885, 3222, 1216, 50, 91, 115, 94, 50, 91, 115, 94],
    [1571, 3609, 427, 3977, 1571, 3609, 427, 3977, 3, 43, 41, 92, 3, 43, 41, 92],
    [2813, 1637, 1479, 2331, 2813, 1637, 1479, 2331, 18, 30, 71, 83, 18, 30, 71, 83],
    [1554, 1648, 3602, 2806, 1554, 1648, 3602, 2806, 94, 25, 3, 3, 94, 25, 3, 3],
    [4073, 1429, 1627, 31, 4073, 1429, 1627, 31, 86, 107, 105, 5, 86, 107, 105, 5],
    [117, 3948, 482, 3509, 117, 3948, 482, 3509, 71, 49, 51, 97, 71, 49, 51, 97],
    [1844, 403, 1628, 2862, 1852, 403, 1628, 2862, 64, 16, 93, 71, 71, 16, 93, 71],
]


def _tc_body(params_ref, x_ref, o_ref):
    ti = jax.lax.broadcasted_iota(jnp.int32, (1, _T), 1)
    fi = jax.lax.broadcasted_iota(jnp.int32, (1, _F), 1)
    for s in range(_NS_TC):
        b = pl.program_id(0) * _NS_TC + s
        mt = jnp.zeros((1, _T), jnp.bool_)
        mf = jnp.zeros((1, _F), jnp.bool_)
        for g in range(_MAXG):
            mt |= (ti >= params_ref[b, g]) & (ti < params_ref[b, _MAXG + g])
            mf |= (fi >= params_ref[b, 2 * _MAXG + g]) & (fi < params_ref[b, 3 * _MAXG + g])
        kt = jnp.where(mt, 0.0, 1.0)
        kf = jnp.where(mf, 0.0, 1.0)
        keep = jax.lax.dot_general(kt, kf, (((0,), (0,)), ((), ())),
                                   preferred_element_type=jnp.float32)
        o_ref[s] = x_ref[s] * keep


def _tc_call(x, params):
    b = x.shape[0]
    return pl.pallas_call(
        _tc_body,
        grid=(b // _NS_TC,),
        in_specs=[
            pl.BlockSpec(memory_space=pltpu.SMEM),
            pl.BlockSpec((_NS_TC, _T, _F), lambda i: (i, 0, 0)),
        ],
        out_specs=pl.BlockSpec((_NS_TC, _T, _F), lambda i: (i, 0, 0)),
        out_shape=jax.ShapeDtypeStruct((b, _T, _F), x.dtype),
        compiler_params=pltpu.CompilerParams(
            dimension_semantics=("parallel",),
            vmem_limit_bytes=120 * 1024 * 1024,
        ),
    )(params, x)


def _sc_body(x_hbm, params_hbm, out_hbm, params_v, buf):
    nsamp = _BS_SC
    wps = _NW // nsamp
    seg_rows = _T // wps
    wid = lax.axis_index("s") * _NC + lax.axis_index("c")
    b = wid // wps
    t_base = (wid % wps) * seg_rows
    pltpu.sync_copy(params_hbm.at[b], params_v)
    pvec = params_v[...]
    ts = [pvec[i] for i in range(_MAXG)]
    te = [pvec[_MAXG + i] for i in range(_MAXG)]
    fs = [pvec[2 * _MAXG + i] for i in range(_MAXG)]
    fe = [pvec[3 * _MAXG + i] for i in range(_MAXG)]
    kf = []
    for g in range(_F // _L):
        fi = lax.iota(jnp.int32, _L) + g * _L
        mf = (fi >= fs[0]) & (fi < fe[0])
        for q in range(1, _MAXG):
            mf |= (fi >= fs[q]) & (fi < fe[q])
        kf.append(jnp.where(mf, jnp.float32(0), jnp.float32(1)))

    def chunk_body(c, carry):
        t0 = t_base + c * _RC
        pltpu.sync_copy(x_hbm.at[b, pl.ds(t0, _RC)], buf)

        def row_body(j, carry2):
            t = t0 + j
            ing = (t >= ts[0]) & (t < te[0])
            for q in range(1, _MAXG):
                ing |= (t >= ts[q]) & (t < te[q])
            kt = jnp.where(ing, jnp.float32(0), jnp.float32(1))
            for g in range(_F // _L):
                v = buf[j, pl.ds(g * _L, _L)]
                buf[j, pl.ds(g * _L, _L)] = v * (kf[g] * kt)
            return carry2

        lax.fori_loop(0, _RC, row_body, 0)
        pltpu.sync_copy(buf, out_hbm.at[b, pl.ds(t0, _RC)])
        return carry

    lax.fori_loop(0, seg_rows // _RC, chunk_body, 0)


def _sc_call(x, params):
    b = x.shape[0]
    call = functools.partial(
        pl.kernel,
        mesh=plsc.VectorSubcoreMesh(core_axis_name="c", subcore_axis_name="s"),
        out_type=jax.ShapeDtypeStruct((b, _T, _F), x.dtype),
        scratch_types=[
            pltpu.VMEM((_L,), jnp.int32),
            pltpu.VMEM((_RC, _F), jnp.float32),
        ],
    )(_sc_body)
    return call(x, params)


def kernel(x):
    params = jnp.asarray(_PARAMS_CONST, dtype=jnp.int32)
    return _tc_call(x, params)
